# baseline (speedup 1.0000x reference)
import functools

import jax
import jax.numpy as jnp
from jax import lax
from jax.experimental import pallas as pl
from jax.experimental.pallas import tpu as pltpu
from jax.experimental.pallas import tpu_sc as plsc

LANES = 16
NXBUF = 3
NPBUF = 2


def _make_sc_kernel(B, S, D):
    info = plsc.get_sparse_core_info()
    NC, NS = info.num_cores, info.num_subcores
    NW = NC * NS
    s_per_w = S // NW
    CH = 16
    n_ch = s_per_w // CH
    n_col = D // LANES
    n_step = n_ch * B

    mesh = plsc.VectorSubcoreMesh(core_axis_name="c", subcore_axis_name="s")

    scratch = (
        [pltpu.VMEM((CH, D), jnp.float32) for _ in range(NXBUF + NPBUF)]
        + [pltpu.SemaphoreType.DMA for _ in range(2 * NXBUF + NPBUF)]
    )

    @functools.partial(
        pl.kernel,
        mesh=mesh,
        out_type=jax.ShapeDtypeStruct((B, S, D), jnp.float32),
        scratch_types=scratch,
        compiler_params=pltpu.CompilerParams(use_tc_tiling_on_sc=True),
    )
    def k(xf, pe, out, xb0, xb1, xb2, pb0, pb1,
          lx0, lx1, lx2, sx0, sx1, sx2, lp0, lp1):
        xbs = (xb0, xb1, xb2)
        pbs = (pb0, pb1)
        lxs = (lx0, lx1, lx2)
        sxs = (sx0, sx1, sx2)
        lps = (lp0, lp1)

        wid = lax.axis_index("s") * NC + lax.axis_index("c")
        s_base = wid * s_per_w

        def load_x(i):
            c, b = divmod(i, B)
            p = i % NXBUF
            s0 = s_base + c * CH
            return pltpu.async_copy(
                xf.at[b, pl.ds(s0, CH), :], xbs[p], lxs[p])

        def load_pe(c):
            q = c % NPBUF
            s0 = s_base + c * CH
            return pltpu.async_copy(pe.at[pl.ds(s0, CH), :], pbs[q], lps[q])

        def store_x(i):
            c, b = divmod(i, B)
            p = i % NXBUF
            s0 = s_base + c * CH
            return pltpu.async_copy(
                xbs[p], out.at[b, pl.ds(s0, CH), :], sxs[p])

        GPB = 16

        def compute(i):
            c = i // B
            xb, pb = xbs[i % NXBUF], pbs[c % NPBUF]

            def rbody(r, carry):
                def cbody(j, carry2):
                    base = j * (GPB * LANES)
                    for g in range(GPB):
                        col = base + g * LANES
                        plsc.addupdate(
                            xb.at[r, pl.ds(col, LANES)],
                            pb[r, pl.ds(col, LANES)])
                    return carry2

                lax.fori_loop(0, n_col // GPB, cbody, 0)
                return carry

            lax.fori_loop(0, CH, rbody, 0)

        pe_loads = {0: load_pe(0), 1: load_pe(1)}
        x_loads = {i: load_x(i) for i in range(min(NXBUF, n_step))}
        stores = {}
        for i in range(n_step):
            c, b = divmod(i, B)
            if i >= NXBUF:
                stores.pop(i - NXBUF).wait()
            if i + NXBUF < n_step:
                x_loads[i + NXBUF] = load_x(i + NXBUF)
            if b == 0:
                pe_loads.pop(c).wait()
            x_loads.pop(i).wait()
            compute(i)
            stores[i] = store_x(i)
            if b == B - 1 and c + 2 < n_ch:
                pe_loads[c + 2] = load_pe(c + 2)
        for h in stores.values():
            h.wait()

    return k


def kernel(x, pe_weight):
    B, S, D = x.shape
    return _make_sc_kernel(B, S, D)(x, pe_weight[:S])
